# baseline (speedup 1.0000x reference)
import functools

import jax
import jax.numpy as jnp
from jax import lax
from jax.experimental import pallas as pl
from jax.experimental.pallas import tpu as pltpu
from jax.experimental.pallas import tpu_sc as plsc

H = 8
F = 16
HF = H * F
DW = 16
NEG_SLOPE = 0.01

_NC = 2
_NS = 16
_NW = _NC * _NS


def _stage_a_body(x_ref, aer_ref, rexp_ref, i8_ref, o_ref, ot_ref):
    x = x_ref[...]
    er = jnp.dot(x, aer_ref[...], preferred_element_type=jnp.float32)
    e = jnp.where(er >= 0, er, er * NEG_SLOPE)
    ee = jnp.exp(e)
    ee128 = jnp.dot(ee, rexp_ref[...], preferred_element_type=jnp.float32)
    o_ref[...] = ee128 * x
    ot_ref[...] = lax.dot_general(
        i8_ref[...], ee, (((1,), (1,)), ((), ())),
        preferred_element_type=jnp.float32)


def _stage_a(feat, aer, rexp, i8, block_e, blk0, nblk):
    ep = nblk * block_e
    return pl.pallas_call(
        _stage_a_body,
        grid=(nblk,),
        in_specs=[
            pl.BlockSpec((block_e, HF), lambda i: (i + blk0, 0)),
            pl.BlockSpec((HF, H), lambda i: (0, 0)),
            pl.BlockSpec((H, HF), lambda i: (0, 0)),
            pl.BlockSpec((H, H), lambda i: (0, 0)),
        ],
        out_specs=[
            pl.BlockSpec((block_e, HF), lambda i: (i, 0)),
            pl.BlockSpec((H, block_e), lambda i: (0, i)),
        ],
        out_shape=[
            jax.ShapeDtypeStruct((ep, HF), jnp.float32),
            jax.ShapeDtypeStruct((H, ep), jnp.float32),
        ],
    )(feat, aer, rexp, i8)


def _stage_b(w, eet, dst, e0, n_pad, init):
    ep = w.shape[0]
    EW = ep // _NW
    C = 80
    NCH = EW // C
    RPS = n_pad // _NS
    ZR = 32
    NZ = RPS // ZR

    mesh = plsc.VectorSubcoreMesh(core_axis_name="c", subcore_axis_name="s")

    @functools.partial(
        pl.kernel,
        mesh=mesh,
        out_type=[
            jax.ShapeDtypeStruct((_NC * n_pad, HF), jnp.float32),
            jax.ShapeDtypeStruct((_NC * n_pad, DW), jnp.float32),
        ],
        compiler_params=pltpu.CompilerParams(
            use_tc_tiling_on_sc=False, needs_layout_passes=False),
        scratch_types=[
            pltpu.VMEM((C, HF), jnp.float32),
            pltpu.VMEM((C, HF), jnp.float32),
            pltpu.VMEM((C,), jnp.int32),
            pltpu.VMEM((C,), jnp.int32),
            pltpu.VMEM((H, C), jnp.float32),
            pltpu.VMEM((H, C), jnp.float32),
            pltpu.VMEM((C, DW), jnp.float32),
            pltpu.VMEM((C, DW), jnp.float32),
            pltpu.SemaphoreType.DMA,
            pltpu.SemaphoreType.DMA,
            pltpu.SemaphoreType.DMA,
            pltpu.SemaphoreType.DMA,
            pltpu.VMEM((ZR, HF), jnp.float32),
            pltpu.VMEM_SHARED((n_pad, HF), jnp.float32),
            pltpu.VMEM_SHARED((n_pad, DW), jnp.float32),
        ],
    )
    def body(*refs):
        if init is None:
            (w_hbm, eet_hbm, dst_hbm, outw_hbm, outd_hbm,
             wv0, wv1, dv0, dv1, ev0, ev1, db0, db1, seml0, seml1,
             sems0, sems1, zbuf, accw, accd) = refs
        else:
            (w_hbm, eet_hbm, dst_hbm, initw_hbm, initd_hbm,
             outw_hbm, outd_hbm,
             wv0, wv1, dv0, dv1, ev0, ev1, db0, db1, seml0, seml1,
             sems0, sems1, zbuf, accw, accd) = refs
        cid = lax.axis_index("c")
        sid = lax.axis_index("s")
        wid = cid * _NS + sid
        rb = sid * RPS
        ob = cid * n_pad + rb
        zero = jnp.zeros((16,), jnp.float32)
        ebase = wid * EW
        bufs = ((wv0, dv0, ev0, db0, seml0, sems0),
                (wv1, dv1, ev1, db1, seml1, sems1))
        lanes = lax.iota(jnp.int32, 16)

        def load(t, b):
            wvb, dvb, evb, dbb, semlb, semsb = bufs[b]
            off = ebase + t * C
            pltpu.async_copy(w_hbm.at[pl.ds(off, C)], wvb, semlb)
            pltpu.async_copy(dst_hbm.at[pl.ds(e0 + off, C)], dvb, semlb)
            pltpu.async_copy(eet_hbm.at[:, pl.ds(off, C)], evb, semlb)

        load(0, 0)

        def dbfill(k, carry):
            db0[k, pl.ds(0, 16)] = zero
            db1[k, pl.ds(0, 16)] = zero
            return carry

        lax.fori_loop(0, C, dbfill, 0)

        if init is None:
            def zfill(k, carry):
                i = k // (HF // 16)
                j = k - i * (HF // 16)
                zbuf[i, pl.ds(j * 16, 16)] = zero
                return carry

            lax.fori_loop(0, ZR * (HF // 16), zfill, 0)
            for q in range(NZ):
                pltpu.sync_copy(zbuf, accw.at[pl.ds(rb + q * ZR, ZR)])
            for q in range(RPS // C):
                pltpu.sync_copy(db0, accd.at[pl.ds(rb + q * C, C)])
        else:
            pltpu.sync_copy(initw_hbm.at[pl.ds(ob, RPS)],
                            accw.at[pl.ds(rb, RPS)])
            pltpu.sync_copy(initd_hbm.at[pl.ds(ob, RPS)],
                            accd.at[pl.ds(rb, RPS)])
        plsc.subcore_barrier()

        def wait_load(t, b):
            wvb, dvb, evb, dbb, semlb, semsb = bufs[b]
            off = ebase + t * C
            pltpu.make_async_copy(w_hbm.at[pl.ds(off, C)], wvb, semlb).wait()
            pltpu.make_async_copy(dst_hbm.at[pl.ds(e0 + off, C)], dvb, semlb).wait()
            pltpu.make_async_copy(eet_hbm.at[:, pl.ds(off, C)], evb, semlb).wait()

        def issue_scatter(b):
            wvb, dvb, evb, dbb, semlb, semsb = bufs[b]
            for h in range(H):
                for g in range(C // 16):
                    v = evb[h, pl.ds(g * 16, 16)]
                    plsc.store_scatter(
                        dbb, [g * 16 + lanes, jnp.full((16,), h, jnp.int32)], v)
            pltpu.async_copy(wvb, accw.at[dvb], semsb, add=True)
            pltpu.async_copy(dbb, accd.at[dvb], semsb, add=True)

        def drain_scatter(b):
            wvb, dvb, evb, dbb, semlb, semsb = bufs[b]
            pltpu.make_async_copy(wvb, accw.at[dvb], semsb).wait()
            pltpu.make_async_copy(dbb, accd.at[dvb], semsb).wait()

        def step(t, b):
            wait_load(t, b)
            issue_scatter(b)

            @pl.when(t >= 1)
            def _():
                drain_scatter(1 - b)

            @pl.when(t + 1 < NCH)
            def _():
                load(t + 1, 1 - b)

        def pair(i, carry):
            t = 2 * i
            step(t, 0)

            @pl.when(t + 1 < NCH)
            def _():
                step(t + 1, 1)

            return carry

        lax.fori_loop(0, (NCH + 1) // 2, pair, 0)
        drain_scatter((NCH - 1) % 2)
        plsc.subcore_barrier()

        pltpu.sync_copy(accw.at[pl.ds(rb, RPS)], outw_hbm.at[pl.ds(ob, RPS)])
        pltpu.sync_copy(accd.at[pl.ds(rb, RPS)], outd_hbm.at[pl.ds(ob, RPS)])

    args = (w, eet, dst) if init is None else (w, eet, dst, init[0], init[1])
    return body(*args)


def _stage_c_body(s_ref, d_ref, rexp_ref, o_ref):
    s = s_ref[0] + s_ref[1]
    den = d_ref[0, :, 0:H] + d_ref[1, :, 0:H]
    dinv = 1.0 / jnp.maximum(den, 1e-9)
    d128 = jnp.dot(dinv, rexp_ref[...], preferred_element_type=jnp.float32)
    v = s * d128
    o_ref[...] = jnp.where(v > 0, v, jnp.exp(v) - 1.0)


def _stage_c(partsw, partsd, rexp, n_nodes, block_n):
    return pl.pallas_call(
        _stage_c_body,
        grid=(n_nodes // block_n,),
        in_specs=[
            pl.BlockSpec((_NC, block_n, HF), lambda i: (0, i, 0)),
            pl.BlockSpec((_NC, block_n, DW), lambda i: (0, i, 0)),
            pl.BlockSpec((H, HF), lambda i: (0, 0)),
        ],
        out_specs=pl.BlockSpec((block_n, HF), lambda i: (i, 0)),
        out_shape=jax.ShapeDtypeStruct((n_nodes, HF), jnp.float32),
    )(partsw, partsd, rexp)


def kernel(feat, attn_r, metapath_idx):
    E = feat.shape[0]
    n_nodes = 10000
    dst = metapath_idx[:, 0].astype(jnp.int32)

    ar = attn_r.reshape(H, F).astype(jnp.float32)
    eye = jnp.eye(H, dtype=jnp.float32)
    aer = (eye[:, :, None] * ar[:, None, :]).transpose(0, 2, 1).reshape(HF, H)
    rexp = jnp.kron(eye, jnp.ones((1, F), jnp.float32))

    n_pad = 10240
    be = 6400
    nblk1 = 30
    nblk2 = (E // be) - nblk1
    e1 = nblk1 * be

    w1, t1 = _stage_a(feat, aer, rexp, eye, be, 0, nblk1)
    w2, t2 = _stage_a(feat, aer, rexp, eye, be, nblk1, nblk2)
    p1 = _stage_b(w1, t1, dst, 0, n_pad, None)
    p2 = _stage_b(w2, t2, dst, e1, n_pad, p1)
    return _stage_c(p2[0].reshape(_NC, n_pad, HF), p2[1].reshape(_NC, n_pad, DW),
                    rexp, n_nodes, block_n=400)


# speedup vs baseline: 1.0603x; 1.0603x over previous
import functools

import jax
import jax.numpy as jnp
from jax import lax
from jax.experimental import pallas as pl
from jax.experimental.pallas import tpu as pltpu
from jax.experimental.pallas import tpu_sc as plsc

H = 8
F = 16
HF = H * F
DW = 16
NEG_SLOPE = 0.01

_NC = 2
_NS = 16
_NW = _NC * _NS


def _stage_a_body(x_ref, aer_ref, rexp_ref, i8_ref, o_ref, ot_ref):
    x = x_ref[...]
    er = jnp.dot(x, aer_ref[...], preferred_element_type=jnp.float32)
    e = jnp.where(er >= 0, er, er * NEG_SLOPE)
    ee = jnp.exp(e)
    ee128 = jnp.dot(ee, rexp_ref[...], preferred_element_type=jnp.float32)
    o_ref[...] = ee128 * x
    ot_ref[...] = lax.dot_general(
        i8_ref[...], ee, (((1,), (1,)), ((), ())),
        preferred_element_type=jnp.float32)


def _stage_a(feat, aer, rexp, i8, block_e, blk0, nblk):
    ep = nblk * block_e
    return pl.pallas_call(
        _stage_a_body,
        grid=(nblk,),
        in_specs=[
            pl.BlockSpec((block_e, HF), lambda i: (i + blk0, 0)),
            pl.BlockSpec((HF, H), lambda i: (0, 0)),
            pl.BlockSpec((H, HF), lambda i: (0, 0)),
            pl.BlockSpec((H, H), lambda i: (0, 0)),
        ],
        out_specs=[
            pl.BlockSpec((block_e, HF), lambda i: (i, 0)),
            pl.BlockSpec((H, block_e), lambda i: (0, i)),
        ],
        out_shape=[
            jax.ShapeDtypeStruct((ep, HF), jnp.float32),
            jax.ShapeDtypeStruct((H, ep), jnp.float32),
        ],
    )(feat, aer, rexp, i8)


def _stage_b(w, eet, dst, e0, n_pad, init):
    ep = w.shape[0]
    EW = ep // _NW
    C = 80
    NCH = EW // C
    RPS = n_pad // _NS
    ZR = 32
    NZ = RPS // ZR

    mesh = plsc.VectorSubcoreMesh(core_axis_name="c", subcore_axis_name="s")

    @functools.partial(
        pl.kernel,
        mesh=mesh,
        out_type=[
            jax.ShapeDtypeStruct((_NC * n_pad, HF), jnp.float32),
            jax.ShapeDtypeStruct((_NC * n_pad, DW), jnp.float32),
        ],
        compiler_params=pltpu.CompilerParams(
            use_tc_tiling_on_sc=False, needs_layout_passes=False),
        scratch_types=[
            pltpu.VMEM((C, HF), jnp.float32),
            pltpu.VMEM((C, HF), jnp.float32),
            pltpu.VMEM((C,), jnp.int32),
            pltpu.VMEM((C,), jnp.int32),
            pltpu.VMEM((H, C), jnp.float32),
            pltpu.VMEM((H, C), jnp.float32),
            pltpu.VMEM((C, DW), jnp.float32),
            pltpu.VMEM((C, DW), jnp.float32),
            pltpu.SemaphoreType.DMA,
            pltpu.SemaphoreType.DMA,
            pltpu.SemaphoreType.DMA,
            pltpu.SemaphoreType.DMA,
            pltpu.VMEM((ZR, HF), jnp.float32),
            pltpu.VMEM_SHARED((n_pad, HF), jnp.float32),
            pltpu.VMEM_SHARED((n_pad, DW), jnp.float32),
        ],
    )
    def body(*refs):
        if init is None:
            (w_hbm, eet_hbm, dst_hbm, outw_hbm, outd_hbm,
             wv0, wv1, dv0, dv1, ev0, ev1, db0, db1, seml0, seml1,
             sems0, sems1, zbuf, accw, accd) = refs
        else:
            (w_hbm, eet_hbm, dst_hbm, initw_hbm, initd_hbm,
             outw_hbm, outd_hbm,
             wv0, wv1, dv0, dv1, ev0, ev1, db0, db1, seml0, seml1,
             sems0, sems1, zbuf, accw, accd) = refs
        cid = lax.axis_index("c")
        sid = lax.axis_index("s")
        wid = cid * _NS + sid
        rb = sid * RPS
        ob = cid * n_pad + rb
        zero = jnp.zeros((16,), jnp.float32)
        ebase = wid * EW
        bufs = ((wv0, dv0, ev0, db0, seml0, sems0),
                (wv1, dv1, ev1, db1, seml1, sems1))
        lanes = lax.iota(jnp.int32, 16)

        def load(t, b):
            wvb, dvb, evb, dbb, semlb, semsb = bufs[b]
            off = ebase + t * C
            pltpu.async_copy(w_hbm.at[pl.ds(off, C)], wvb, semlb)
            pltpu.async_copy(dst_hbm.at[pl.ds(e0 + off, C)], dvb, semlb)
            pltpu.async_copy(eet_hbm.at[:, pl.ds(off, C)], evb, semlb)

        load(0, 0)

        def dbfill(k, carry):
            db0[k, pl.ds(0, 16)] = zero
            db1[k, pl.ds(0, 16)] = zero
            return carry

        lax.fori_loop(0, C, dbfill, 0)

        if init is None:
            def zfill(k, carry):
                i = k // (HF // 16)
                j = k - i * (HF // 16)
                zbuf[i, pl.ds(j * 16, 16)] = zero
                return carry

            lax.fori_loop(0, ZR * (HF // 16), zfill, 0)
            for q in range(NZ):
                pltpu.sync_copy(zbuf, accw.at[pl.ds(rb + q * ZR, ZR)])
            for q in range(RPS // C):
                pltpu.sync_copy(db0, accd.at[pl.ds(rb + q * C, C)])
        else:
            pltpu.sync_copy(initw_hbm.at[pl.ds(ob, RPS)],
                            accw.at[pl.ds(rb, RPS)])
            pltpu.sync_copy(initd_hbm.at[pl.ds(ob, RPS)],
                            accd.at[pl.ds(rb, RPS)])
        plsc.subcore_barrier()

        def wait_load(t, b):
            wvb, dvb, evb, dbb, semlb, semsb = bufs[b]
            off = ebase + t * C
            pltpu.make_async_copy(w_hbm.at[pl.ds(off, C)], wvb, semlb).wait()
            pltpu.make_async_copy(dst_hbm.at[pl.ds(e0 + off, C)], dvb, semlb).wait()
            pltpu.make_async_copy(eet_hbm.at[:, pl.ds(off, C)], evb, semlb).wait()

        def step(t, b):
            wait_load(t, b)

            @pl.when(t + 1 < NCH)
            def _():
                load(t + 1, 1 - b)

            wvb, dvb, evb, dbb, semlb, semsb = bufs[b]
            for h in range(H):
                for g in range(C // 16):
                    v = evb[h, pl.ds(g * 16, 16)]
                    plsc.store_scatter(
                        dbb, [g * 16 + lanes, jnp.full((16,), h, jnp.int32)], v)
            pltpu.sync_copy(wvb, accw.at[dvb], add=True)
            pltpu.sync_copy(dbb, accd.at[dvb], add=True)

        def pair(i, carry):
            t = 2 * i
            step(t, 0)

            @pl.when(t + 1 < NCH)
            def _():
                step(t + 1, 1)

            return carry

        lax.fori_loop(0, (NCH + 1) // 2, pair, 0)
        plsc.subcore_barrier()

        pltpu.sync_copy(accw.at[pl.ds(rb, RPS)], outw_hbm.at[pl.ds(ob, RPS)])
        pltpu.sync_copy(accd.at[pl.ds(rb, RPS)], outd_hbm.at[pl.ds(ob, RPS)])

    args = (w, eet, dst) if init is None else (w, eet, dst, init[0], init[1])
    return body(*args)


def _stage_c_body(s_ref, d_ref, rexp_ref, o_ref):
    s = s_ref[0] + s_ref[1]
    den = d_ref[0, :, 0:H] + d_ref[1, :, 0:H]
    dinv = 1.0 / jnp.maximum(den, 1e-9)
    d128 = jnp.dot(dinv, rexp_ref[...], preferred_element_type=jnp.float32)
    v = s * d128
    o_ref[...] = jnp.where(v > 0, v, jnp.exp(v) - 1.0)


def _stage_c(partsw, partsd, rexp, n_nodes, block_n):
    return pl.pallas_call(
        _stage_c_body,
        grid=(n_nodes // block_n,),
        in_specs=[
            pl.BlockSpec((_NC, block_n, HF), lambda i: (0, i, 0)),
            pl.BlockSpec((_NC, block_n, DW), lambda i: (0, i, 0)),
            pl.BlockSpec((H, HF), lambda i: (0, 0)),
        ],
        out_specs=pl.BlockSpec((block_n, HF), lambda i: (i, 0)),
        out_shape=jax.ShapeDtypeStruct((n_nodes, HF), jnp.float32),
    )(partsw, partsd, rexp)


def kernel(feat, attn_r, metapath_idx):
    E = feat.shape[0]
    n_nodes = 10000
    dst = metapath_idx[:, 0].astype(jnp.int32)

    ar = attn_r.reshape(H, F).astype(jnp.float32)
    eye = jnp.eye(H, dtype=jnp.float32)
    aer = (eye[:, :, None] * ar[:, None, :]).transpose(0, 2, 1).reshape(HF, H)
    rexp = jnp.kron(eye, jnp.ones((1, F), jnp.float32))

    n_pad = 10240
    be = 6400
    nblk1 = 30
    nblk2 = (E // be) - nblk1
    e1 = nblk1 * be

    w1, t1 = _stage_a(feat, aer, rexp, eye, be, 0, nblk1)
    w2, t2 = _stage_a(feat, aer, rexp, eye, be, nblk1, nblk2)
    p1 = _stage_b(w1, t1, dst, 0, n_pad, None)
    p2 = _stage_b(w2, t2, dst, e1, n_pad, p1)
    return _stage_c(p2[0].reshape(_NC, n_pad, HF), p2[1].reshape(_NC, n_pad, DW),
                    rexp, n_nodes, block_n=400)


# speedup vs baseline: 1.1393x; 1.0745x over previous
import functools

import jax
import jax.numpy as jnp
from jax import lax
from jax.experimental import pallas as pl
from jax.experimental.pallas import tpu as pltpu
from jax.experimental.pallas import tpu_sc as plsc

H = 8
F = 16
HF = H * F
DW = 16
NEG_SLOPE = 0.01

_NC = 2
_NS = 16
_NW = _NC * _NS


def _stage_a_body(x_ref, aer_ref, rexp_ref, i8_ref, o_ref, ot_ref):
    x = x_ref[...]
    er = jnp.dot(x, aer_ref[...], preferred_element_type=jnp.float32)
    e = jnp.where(er >= 0, er, er * NEG_SLOPE)
    ee = jnp.exp(e)
    ee128 = jnp.dot(ee, rexp_ref[...], preferred_element_type=jnp.float32)
    o_ref[...] = ee128 * x
    ot_ref[...] = lax.dot_general(
        i8_ref[...], ee, (((1,), (1,)), ((), ())),
        preferred_element_type=jnp.float32)


def _stage_a(feat, aer, rexp, i8, block_e, blk0, nblk):
    ep = nblk * block_e
    return pl.pallas_call(
        _stage_a_body,
        grid=(nblk,),
        in_specs=[
            pl.BlockSpec((block_e, HF), lambda i: (i + blk0, 0)),
            pl.BlockSpec((HF, H), lambda i: (0, 0)),
            pl.BlockSpec((H, HF), lambda i: (0, 0)),
            pl.BlockSpec((H, H), lambda i: (0, 0)),
        ],
        out_specs=[
            pl.BlockSpec((block_e, HF), lambda i: (i, 0)),
            pl.BlockSpec((H, block_e), lambda i: (0, i)),
        ],
        out_shape=[
            jax.ShapeDtypeStruct((ep, HF), jnp.float32),
            jax.ShapeDtypeStruct((H, ep), jnp.float32),
        ],
    )(feat, aer, rexp, i8)


def _stage_b(w, eet, dst, e0, n_pad, init):
    ep = w.shape[0]
    EW = ep // _NW
    C = 80
    NCH = EW // C
    RPS = n_pad // _NS
    ZR = 32
    NZ = RPS // ZR

    mesh = plsc.VectorSubcoreMesh(core_axis_name="c", subcore_axis_name="s")

    @functools.partial(
        pl.kernel,
        mesh=mesh,
        out_type=[
            jax.ShapeDtypeStruct((_NC * n_pad, HF), jnp.float32),
            jax.ShapeDtypeStruct((_NC * n_pad, DW), jnp.float32),
        ],
        compiler_params=pltpu.CompilerParams(
            use_tc_tiling_on_sc=False, needs_layout_passes=False),
        scratch_types=[
            pltpu.VMEM((C, HF), jnp.float32),
            pltpu.VMEM((C, HF), jnp.float32),
            pltpu.VMEM((C,), jnp.int32),
            pltpu.VMEM((C,), jnp.int32),
            pltpu.VMEM((H, C), jnp.float32),
            pltpu.VMEM((H, C), jnp.float32),
            pltpu.VMEM((C, DW), jnp.float32),
            pltpu.VMEM((C, DW), jnp.float32),
            pltpu.SemaphoreType.DMA,
            pltpu.SemaphoreType.DMA,
            pltpu.SemaphoreType.DMA,
            pltpu.SemaphoreType.DMA,
            pltpu.VMEM((ZR, HF), jnp.float32),
            pltpu.VMEM_SHARED((n_pad, HF), jnp.float32),
            pltpu.VMEM_SHARED((n_pad, DW), jnp.float32),
        ],
    )
    def body(*refs):
        if init is None:
            (w_hbm, eet_hbm, dst_hbm, outw_hbm, outd_hbm,
             wv0, wv1, dv0, dv1, ev0, ev1, db0, db1, seml0, seml1,
             sems0, sems1, zbuf, accw, accd) = refs
        else:
            (w_hbm, eet_hbm, dst_hbm, initw_hbm, initd_hbm,
             outw_hbm, outd_hbm,
             wv0, wv1, dv0, dv1, ev0, ev1, db0, db1, seml0, seml1,
             sems0, sems1, zbuf, accw, accd) = refs
        cid = lax.axis_index("c")
        sid = lax.axis_index("s")
        wid = cid * _NS + sid
        rb = sid * RPS
        ob = cid * n_pad + rb
        zero = jnp.zeros((16,), jnp.float32)
        ebase = wid * EW
        bufs = ((wv0, dv0, ev0, db0, seml0, sems0),
                (wv1, dv1, ev1, db1, seml1, sems1))
        lanes = lax.iota(jnp.int32, 16)

        def load(t, b):
            wvb, dvb, evb, dbb, semlb, semsb = bufs[b]
            off = ebase + t * C
            pltpu.async_copy(w_hbm.at[pl.ds(off, C)], wvb, semlb)
            pltpu.async_copy(dst_hbm.at[pl.ds(e0 + off, C)], dvb, semlb)
            pltpu.async_copy(eet_hbm.at[:, pl.ds(off, C)], evb, semlb)

        load(0, 0)

        def dbfill(k, carry):
            db0[k, pl.ds(0, 16)] = zero
            db1[k, pl.ds(0, 16)] = zero
            return carry

        lax.fori_loop(0, C, dbfill, 0)

        if init is None:
            def zfill(k, carry):
                i = k // (HF // 16)
                j = k - i * (HF // 16)
                zbuf[i, pl.ds(j * 16, 16)] = zero
                return carry

            lax.fori_loop(0, ZR * (HF // 16), zfill, 0)
            for q in range(NZ):
                pltpu.sync_copy(zbuf, accw.at[pl.ds(rb + q * ZR, ZR)])
            for q in range(RPS // C):
                pltpu.sync_copy(db0, accd.at[pl.ds(rb + q * C, C)])
        else:
            pltpu.sync_copy(initw_hbm.at[pl.ds(ob, RPS)],
                            accw.at[pl.ds(rb, RPS)])
            pltpu.sync_copy(initd_hbm.at[pl.ds(ob, RPS)],
                            accd.at[pl.ds(rb, RPS)])
        plsc.subcore_barrier()

        def wait_load(t, b):
            wvb, dvb, evb, dbb, semlb, semsb = bufs[b]
            off = ebase + t * C
            pltpu.make_async_copy(w_hbm.at[pl.ds(off, C)], wvb, semlb).wait()
            pltpu.make_async_copy(dst_hbm.at[pl.ds(e0 + off, C)], dvb, semlb).wait()
            pltpu.make_async_copy(eet_hbm.at[:, pl.ds(off, C)], evb, semlb).wait()

        def step(t, b):
            @pl.when(t + 1 < NCH)
            def _():
                load(t + 1, 1 - b)

            wait_load(t, b)
            wvb, dvb, evb, dbb, semlb, semsb = bufs[b]
            for h in range(H):
                for g in range(C // 16):
                    v = evb[h, pl.ds(g * 16, 16)]
                    plsc.store_scatter(
                        dbb, [g * 16 + lanes, jnp.full((16,), h, jnp.int32)], v)
            pltpu.sync_copy(wvb, accw.at[dvb], add=True)
            pltpu.sync_copy(dbb, accd.at[dvb], add=True)

        def pair(i, carry):
            t = 2 * i
            step(t, 0)

            @pl.when(t + 1 < NCH)
            def _():
                step(t + 1, 1)

            return carry

        lax.fori_loop(0, (NCH + 1) // 2, pair, 0)
        plsc.subcore_barrier()

        pltpu.sync_copy(accw.at[pl.ds(rb, RPS)], outw_hbm.at[pl.ds(ob, RPS)])
        pltpu.sync_copy(accd.at[pl.ds(rb, RPS)], outd_hbm.at[pl.ds(ob, RPS)])

    args = (w, eet, dst) if init is None else (w, eet, dst, init[0], init[1])
    return body(*args)


def _stage_c_body(s_ref, d_ref, rexp_ref, o_ref):
    s = s_ref[0] + s_ref[1]
    den = d_ref[0, :, 0:H] + d_ref[1, :, 0:H]
    dinv = 1.0 / jnp.maximum(den, 1e-9)
    d128 = jnp.dot(dinv, rexp_ref[...], preferred_element_type=jnp.float32)
    v = s * d128
    o_ref[...] = jnp.where(v > 0, v, jnp.exp(v) - 1.0)


def _stage_c(partsw, partsd, rexp, n_nodes, block_n):
    return pl.pallas_call(
        _stage_c_body,
        grid=(n_nodes // block_n,),
        in_specs=[
            pl.BlockSpec((_NC, block_n, HF), lambda i: (0, i, 0)),
            pl.BlockSpec((_NC, block_n, DW), lambda i: (0, i, 0)),
            pl.BlockSpec((H, HF), lambda i: (0, 0)),
        ],
        out_specs=pl.BlockSpec((block_n, HF), lambda i: (i, 0)),
        out_shape=jax.ShapeDtypeStruct((n_nodes, HF), jnp.float32),
    )(partsw, partsd, rexp)


def kernel(feat, attn_r, metapath_idx):
    E = feat.shape[0]
    n_nodes = 10000
    dst = metapath_idx[:, 0].astype(jnp.int32)

    ar = attn_r.reshape(H, F).astype(jnp.float32)
    eye = jnp.eye(H, dtype=jnp.float32)
    aer = (eye[:, :, None] * ar[:, None, :]).transpose(0, 2, 1).reshape(HF, H)
    rexp = jnp.kron(eye, jnp.ones((1, F), jnp.float32))

    n_pad = 10240
    be = 6400
    nblk1 = 30
    nblk2 = (E // be) - nblk1
    e1 = nblk1 * be

    w1, t1 = _stage_a(feat, aer, rexp, eye, be, 0, nblk1)
    w2, t2 = _stage_a(feat, aer, rexp, eye, be, nblk1, nblk2)
    p1 = _stage_b(w1, t1, dst, 0, n_pad, None)
    p2 = _stage_b(w2, t2, dst, e1, n_pad, p1)
    return _stage_c(p2[0].reshape(_NC, n_pad, HF), p2[1].reshape(_NC, n_pad, DW),
                    rexp, n_nodes, block_n=400)


# speedup vs baseline: 1.1932x; 1.0473x over previous
import functools

import jax
import jax.numpy as jnp
from jax import lax
from jax.experimental import pallas as pl
from jax.experimental.pallas import tpu as pltpu
from jax.experimental.pallas import tpu_sc as plsc

H = 8
F = 16
HF = H * F
DW = 16
NEG_SLOPE = 0.01

_NC = 2
_NS = 16
_NW = _NC * _NS


def _stage_a_body(x_ref, aer_ref, rexp_ref, i8_ref, o_ref, ot_ref):
    x = x_ref[...]
    er = jnp.dot(x, aer_ref[...], preferred_element_type=jnp.float32)
    e = jnp.where(er >= 0, er, er * NEG_SLOPE)
    ee = jnp.exp(e)
    ee128 = jnp.dot(ee, rexp_ref[...], preferred_element_type=jnp.float32)
    o_ref[...] = ee128 * x
    ot_ref[...] = lax.dot_general(
        i8_ref[...], ee, (((1,), (1,)), ((), ())),
        preferred_element_type=jnp.float32)


def _stage_a(feat, aer, rexp, i8, block_e, blk0, nblk):
    ep = nblk * block_e
    return pl.pallas_call(
        _stage_a_body,
        grid=(nblk,),
        in_specs=[
            pl.BlockSpec((block_e, HF), lambda i: (i + blk0, 0)),
            pl.BlockSpec((HF, H), lambda i: (0, 0)),
            pl.BlockSpec((H, HF), lambda i: (0, 0)),
            pl.BlockSpec((H, H), lambda i: (0, 0)),
        ],
        out_specs=[
            pl.BlockSpec((block_e, HF), lambda i: (i, 0)),
            pl.BlockSpec((H, block_e), lambda i: (0, i)),
        ],
        out_shape=[
            jax.ShapeDtypeStruct((ep, HF), jnp.float32),
            jax.ShapeDtypeStruct((H, ep), jnp.float32),
        ],
    )(feat, aer, rexp, i8)


def _stage_b(w, eet, dst, e0, n_pad, init):
    ep = w.shape[0]
    EW = ep // _NW
    C = 80
    NCH = EW // C
    RPS = n_pad // _NS
    ZR = 32
    NZ = RPS // ZR

    mesh = plsc.VectorSubcoreMesh(core_axis_name="c", subcore_axis_name="s")

    @functools.partial(
        pl.kernel,
        mesh=mesh,
        out_type=[
            jax.ShapeDtypeStruct((_NC * n_pad, HF), jnp.float32),
            jax.ShapeDtypeStruct((_NC * n_pad, DW), jnp.float32),
        ],
        compiler_params=pltpu.CompilerParams(
            use_tc_tiling_on_sc=False, needs_layout_passes=False),
        scratch_types=[
            pltpu.VMEM((C, HF), jnp.float32),
            pltpu.VMEM((C, HF), jnp.float32),
            pltpu.VMEM((C,), jnp.int32),
            pltpu.VMEM((C,), jnp.int32),
            pltpu.VMEM((H, C), jnp.float32),
            pltpu.VMEM((H, C), jnp.float32),
            pltpu.VMEM((C, DW), jnp.float32),
            pltpu.VMEM((C, DW), jnp.float32),
            pltpu.SemaphoreType.DMA,
            pltpu.SemaphoreType.DMA,
            pltpu.SemaphoreType.DMA,
            pltpu.SemaphoreType.DMA,
            pltpu.VMEM((ZR, HF), jnp.float32),
            pltpu.VMEM_SHARED((n_pad, HF), jnp.float32),
            pltpu.VMEM_SHARED((n_pad, DW), jnp.float32),
        ],
    )
    def body(*refs):
        if init is None:
            (w_hbm, eet_hbm, dst_hbm, outw_hbm, outd_hbm,
             wv0, wv1, dv0, dv1, ev0, ev1, db0, db1, seml0, seml1,
             sems0, sems1, zbuf, accw, accd) = refs
        else:
            (w_hbm, eet_hbm, dst_hbm, initw_hbm, initd_hbm,
             outw_hbm, outd_hbm,
             wv0, wv1, dv0, dv1, ev0, ev1, db0, db1, seml0, seml1,
             sems0, sems1, zbuf, accw, accd) = refs
        cid = lax.axis_index("c")
        sid = lax.axis_index("s")
        wid = cid * _NS + sid
        rb = sid * RPS
        ob = cid * n_pad + rb
        zero = jnp.zeros((16,), jnp.float32)
        ebase = wid * EW
        bufs = ((wv0, dv0, ev0, db0, seml0, sems0),
                (wv1, dv1, ev1, db1, seml1, sems1))
        lanes = lax.iota(jnp.int32, 16)

        def load(t, b):
            wvb, dvb, evb, dbb, semlb, semsb = bufs[b]
            off = ebase + t * C
            pltpu.async_copy(w_hbm.at[pl.ds(off, C)], wvb, semlb)
            pltpu.async_copy(dst_hbm.at[pl.ds(e0 + off, C)], dvb, semlb)
            pltpu.async_copy(eet_hbm.at[:, pl.ds(off, C)], evb, semlb)

        load(0, 0)

        def dbfill(k, carry):
            db0[k, pl.ds(0, 16)] = zero
            db1[k, pl.ds(0, 16)] = zero
            return carry

        lax.fori_loop(0, C, dbfill, 0)

        if init is None:
            def zfill(k, carry):
                i = k // (HF // 16)
                j = k - i * (HF // 16)
                zbuf[i, pl.ds(j * 16, 16)] = zero
                return carry

            lax.fori_loop(0, ZR * (HF // 16), zfill, 0)
            for q in range(NZ):
                pltpu.sync_copy(zbuf, accw.at[pl.ds(rb + q * ZR, ZR)])
            for q in range(RPS // C):
                pltpu.sync_copy(db0, accd.at[pl.ds(rb + q * C, C)])
        else:
            pltpu.sync_copy(initw_hbm.at[pl.ds(ob, RPS)],
                            accw.at[pl.ds(rb, RPS)])
            pltpu.sync_copy(initd_hbm.at[pl.ds(ob, RPS)],
                            accd.at[pl.ds(rb, RPS)])
        plsc.subcore_barrier()

        def wait_load(t, b):
            wvb, dvb, evb, dbb, semlb, semsb = bufs[b]
            off = ebase + t * C
            pltpu.make_async_copy(w_hbm.at[pl.ds(off, C)], wvb, semlb).wait()
            pltpu.make_async_copy(dst_hbm.at[pl.ds(e0 + off, C)], dvb, semlb).wait()
            pltpu.make_async_copy(eet_hbm.at[:, pl.ds(off, C)], evb, semlb).wait()

        def step(t, b):
            @pl.when(t + 1 < NCH)
            def _():
                load(t + 1, 1 - b)

            wait_load(t, b)
            wvb, dvb, evb, dbb, semlb, semsb = bufs[b]
            for h in range(H):
                for g in range(C // 16):
                    v = evb[h, pl.ds(g * 16, 16)]
                    plsc.store_scatter(
                        dbb, [g * 16 + lanes, jnp.full((16,), h, jnp.int32)], v)
            pltpu.sync_copy(wvb, accw.at[dvb], add=True)
            pltpu.sync_copy(dbb, accd.at[dvb], add=True)

        def pair(i, carry):
            t = 2 * i
            step(t, 0)

            @pl.when(t + 1 < NCH)
            def _():
                step(t + 1, 1)

            return carry

        lax.fori_loop(0, (NCH + 1) // 2, pair, 0)
        plsc.subcore_barrier()

        pltpu.sync_copy(accw.at[pl.ds(rb, RPS)], outw_hbm.at[pl.ds(ob, RPS)])
        pltpu.sync_copy(accd.at[pl.ds(rb, RPS)], outd_hbm.at[pl.ds(ob, RPS)])

    args = (w, eet, dst) if init is None else (w, eet, dst, init[0], init[1])
    return body(*args)


def _stage_c_body(s_ref, d_ref, rexp_ref, o_ref):
    s = s_ref[0] + s_ref[1]
    den = d_ref[0, :, 0:H] + d_ref[1, :, 0:H]
    dinv = 1.0 / jnp.maximum(den, 1e-9)
    d128 = jnp.dot(dinv, rexp_ref[...], preferred_element_type=jnp.float32)
    v = s * d128
    o_ref[...] = jnp.where(v > 0, v, jnp.exp(v) - 1.0)


def _stage_c(partsw, partsd, rexp, n_nodes, block_n):
    return pl.pallas_call(
        _stage_c_body,
        grid=(n_nodes // block_n,),
        in_specs=[
            pl.BlockSpec((_NC, block_n, HF), lambda i: (0, i, 0)),
            pl.BlockSpec((_NC, block_n, DW), lambda i: (0, i, 0)),
            pl.BlockSpec((H, HF), lambda i: (0, 0)),
        ],
        out_specs=pl.BlockSpec((block_n, HF), lambda i: (i, 0)),
        out_shape=jax.ShapeDtypeStruct((n_nodes, HF), jnp.float32),
    )(partsw, partsd, rexp)


def kernel(feat, attn_r, metapath_idx):
    E = feat.shape[0]
    n_nodes = 10000
    dst = metapath_idx[:, 0].astype(jnp.int32)

    ar = attn_r.reshape(H, F).astype(jnp.float32)
    eye = jnp.eye(H, dtype=jnp.float32)
    aer = (eye[:, :, None] * ar[:, None, :]).transpose(0, 2, 1).reshape(HF, H)
    rexp = jnp.kron(eye, jnp.ones((1, F), jnp.float32))

    n_pad = 10240
    be = 6400
    nblk1 = 20
    nblk2 = (E // be) - nblk1
    e1 = nblk1 * be

    w1, t1 = _stage_a(feat, aer, rexp, eye, be, 0, nblk1)
    w2, t2 = _stage_a(feat, aer, rexp, eye, be, nblk1, nblk2)
    p1 = _stage_b(w1, t1, dst, 0, n_pad, None)
    p2 = _stage_b(w2, t2, dst, e1, n_pad, p1)
    return _stage_c(p2[0].reshape(_NC, n_pad, HF), p2[1].reshape(_NC, n_pad, DW),
                    rexp, n_nodes, block_n=400)
